# Initial kernel scaffold; baseline (speedup 1.0000x reference)
import jax
import jax.numpy as jnp
from jax.experimental import pallas as pl


def kernel(x_e, edge_index, h_v, c_v, v_batch, W_ih, W_hh, b_ih, b_hh):
    raise NotImplementedError("write your pallas kernel here")


# speedup vs baseline: 4.3800x; 4.3800x over previous
import functools

import jax
import jax.numpy as jnp
from jax import lax
from jax.experimental import pallas as pl
from jax.experimental.pallas import tpu as pltpu
from jax.experimental.pallas import tpu_sc as plsc

V = 10000
E = 320000
D = 128

NC = 2
NS = 16
NW = NC * NS

EPC = E // NC
EPW = E // NW
CHUNK = 80
NCHUNK = EPW // CHUNK
VP = 10240
RPT = VP // NS

_mesh = plsc.VectorSubcoreMesh(core_axis_name="c", subcore_axis_name="s")


@functools.partial(
    pl.kernel,
    out_type=jax.ShapeDtypeStruct((NC, VP, D), jnp.float32),
    mesh=_mesh,
    scratch_types=[
        pltpu.VMEM((CHUNK, D), jnp.float32),
        pltpu.VMEM((CHUNK,), jnp.int32),
        pltpu.VMEM((CHUNK,), jnp.int32),
        pltpu.VMEM_SHARED((VP, D), jnp.float32),
    ],
)
def _scatter_add_sc(x_hbm, src_hbm, dst_hbm, zeros_hbm, out_hbm,
                    xbuf, idx0, idx1, msg_sh):
    c = lax.axis_index("c")
    s = lax.axis_index("s")
    base = c * EPC + s * EPW

    pltpu.sync_copy(zeros_hbm.at[pl.ds(s * RPT, RPT)],
                    msg_sh.at[pl.ds(s * RPT, RPT)])
    plsc.subcore_barrier()

    def body(j, carry):
        e0 = base + j * CHUNK
        pltpu.sync_copy(x_hbm.at[pl.ds(e0, CHUNK)], xbuf)
        pltpu.sync_copy(src_hbm.at[pl.ds(e0, CHUNK)], idx0)
        pltpu.sync_copy(dst_hbm.at[pl.ds(e0, CHUNK)], idx1)
        pltpu.sync_copy(xbuf, msg_sh.at[idx0], add=True)
        pltpu.sync_copy(xbuf, msg_sh.at[idx1], add=True)
        return carry

    lax.fori_loop(0, NCHUNK, body, 0)
    plsc.subcore_barrier()

    pltpu.sync_copy(msg_sh.at[pl.ds(s * RPT, RPT)],
                    out_hbm.at[c, pl.ds(s * RPT, RPT)])


BLK = 400


def _lstm_body(p_ref, h_ref, c_ref, wih_ref, whh_ref, b_ref, ho_ref, co_ref):
    msg = p_ref[0] + p_ref[1]
    h = h_ref[...]
    gates = lax.dot_general(msg, wih_ref[...], (((1,), (1,)), ((), ())),
                            preferred_element_type=jnp.float32)
    gates = gates + lax.dot_general(h, whh_ref[...], (((1,), (1,)), ((), ())),
                                    preferred_element_type=jnp.float32)
    gates = gates + b_ref[...]
    i = jax.nn.sigmoid(gates[:, 0 * D:1 * D])
    f = jax.nn.sigmoid(gates[:, 1 * D:2 * D])
    g = jnp.tanh(gates[:, 2 * D:3 * D])
    o = jax.nn.sigmoid(gates[:, 3 * D:4 * D])
    c_new = f * c_ref[...] + i * g
    ho_ref[...] = o * jnp.tanh(c_new)
    co_ref[...] = c_new


_lstm_call = pl.pallas_call(
    _lstm_body,
    grid=(V // BLK,),
    in_specs=[
        pl.BlockSpec((NC, BLK, D), lambda i: (0, i, 0)),
        pl.BlockSpec((BLK, D), lambda i: (i, 0)),
        pl.BlockSpec((BLK, D), lambda i: (i, 0)),
        pl.BlockSpec((4 * D, D), lambda i: (0, 0)),
        pl.BlockSpec((4 * D, D), lambda i: (0, 0)),
        pl.BlockSpec((1, 4 * D), lambda i: (0, 0)),
    ],
    out_specs=[
        pl.BlockSpec((BLK, D), lambda i: (i, 0)),
        pl.BlockSpec((BLK, D), lambda i: (i, 0)),
    ],
    out_shape=[
        jax.ShapeDtypeStruct((V, D), jnp.float32),
        jax.ShapeDtypeStruct((V, D), jnp.float32),
    ],
)


@jax.jit
def kernel(x_e, edge_index, h_v, c_v, v_batch, W_ih, W_hh, b_ih, b_hh):
    del v_batch
    src = edge_index[0].astype(jnp.int32)
    dst = edge_index[1].astype(jnp.int32)
    zeros = jnp.zeros((VP, D), jnp.float32)
    partials = _scatter_add_sc(x_e, src, dst, zeros)
    bias = (b_ih + b_hh).reshape(1, 4 * D)
    h_new, c_new = _lstm_call(partials, h_v, c_v, W_ih, W_hh, bias)
    return (h_new, c_new)


# speedup vs baseline: 8.8475x; 2.0200x over previous
import functools

import jax
import jax.numpy as jnp
from jax import lax
from jax.experimental import pallas as pl
from jax.experimental.pallas import tpu as pltpu
from jax.experimental.pallas import tpu_sc as plsc

V = 10000
E = 320000
D = 128

NC = 2
NS = 16
NW = NC * NS

EPC = E // NC
EPW = E // NW
CHUNK = 40
NBUF = 5
GROUP = NBUF * CHUNK
NGRP = EPW // GROUP
VP = 10240
RPT = VP // NS

_mesh = plsc.VectorSubcoreMesh(core_axis_name="c", subcore_axis_name="s")


@functools.partial(
    pl.kernel,
    out_type=jax.ShapeDtypeStruct((NC, VP, D), jnp.float32),
    mesh=_mesh,
    scratch_types=(
        [pltpu.VMEM((CHUNK, D), jnp.float32) for _ in range(NBUF)]
        + [pltpu.VMEM((CHUNK,), jnp.int32) for _ in range(2 * NBUF)]
        + [pltpu.VMEM_SHARED((VP, D), jnp.float32)]
        + [pltpu.SemaphoreType.DMA for _ in range(2 * NBUF)]
    ),
)
def _scatter_add_sc(x_hbm, src_hbm, dst_hbm, zeros_hbm, out_hbm, *scratch):
    xb = scratch[0:NBUF]
    i0b = scratch[NBUF:2 * NBUF]
    i1b = scratch[2 * NBUF:3 * NBUF]
    msg_sh = scratch[3 * NBUF]
    lsem = scratch[3 * NBUF + 1:3 * NBUF + 1 + NBUF]
    ssem = scratch[3 * NBUF + 1 + NBUF:3 * NBUF + 1 + 2 * NBUF]

    c = lax.axis_index("c")
    s = lax.axis_index("s")
    base = c * EPC + s * EPW

    def issue_loads(b, e0):
        pltpu.async_copy(x_hbm.at[pl.ds(e0, CHUNK)], xb[b], lsem[b])
        pltpu.async_copy(src_hbm.at[pl.ds(e0, CHUNK)], i0b[b], lsem[b])
        pltpu.async_copy(dst_hbm.at[pl.ds(e0, CHUNK)], i1b[b], lsem[b])

    def wait_loads(b):
        pltpu.make_async_copy(x_hbm.at[pl.ds(0, CHUNK)], xb[b], lsem[b]).wait()
        pltpu.make_async_copy(src_hbm.at[pl.ds(0, CHUNK)], i0b[b], lsem[b]).wait()
        pltpu.make_async_copy(dst_hbm.at[pl.ds(0, CHUNK)], i1b[b], lsem[b]).wait()

    pltpu.sync_copy(zeros_hbm.at[pl.ds(s * RPT, RPT)],
                    msg_sh.at[pl.ds(s * RPT, RPT)])
    plsc.subcore_barrier()

    for b in range(NBUF):
        issue_loads(b, base + b * CHUNK)

    def group(g, carry):
        for b in range(NBUF):
            wait_loads(b)
            pltpu.async_copy(xb[b], msg_sh.at[i0b[b]], ssem[b], add=True)
            pltpu.async_copy(xb[b], msg_sh.at[i1b[b]], ssem[b], add=True)
        for b in range(NBUF):
            pltpu.make_async_copy(xb[b], msg_sh.at[i0b[b]], ssem[b]).wait()
            pltpu.make_async_copy(xb[b], msg_sh.at[i1b[b]], ssem[b]).wait()

            @pl.when(g < NGRP - 1)
            def _():
                issue_loads(b, base + (g + 1) * GROUP + b * CHUNK)
        return carry

    lax.fori_loop(0, NGRP, group, 0)
    plsc.subcore_barrier()

    pltpu.sync_copy(msg_sh.at[pl.ds(s * RPT, RPT)],
                    out_hbm.at[c, pl.ds(s * RPT, RPT)])


BLK = 400


def _lstm_body(p_ref, h_ref, c_ref, wih_ref, whh_ref, b_ref, ho_ref, co_ref):
    msg = p_ref[0] + p_ref[1]
    h = h_ref[...]
    gates = lax.dot_general(msg, wih_ref[...], (((1,), (1,)), ((), ())),
                            preferred_element_type=jnp.float32)
    gates = gates + lax.dot_general(h, whh_ref[...], (((1,), (1,)), ((), ())),
                                    preferred_element_type=jnp.float32)
    gates = gates + b_ref[...]
    i = jax.nn.sigmoid(gates[:, 0 * D:1 * D])
    f = jax.nn.sigmoid(gates[:, 1 * D:2 * D])
    g = jnp.tanh(gates[:, 2 * D:3 * D])
    o = jax.nn.sigmoid(gates[:, 3 * D:4 * D])
    c_new = f * c_ref[...] + i * g
    ho_ref[...] = o * jnp.tanh(c_new)
    co_ref[...] = c_new


_lstm_call = pl.pallas_call(
    _lstm_body,
    grid=(V // BLK,),
    in_specs=[
        pl.BlockSpec((NC, BLK, D), lambda i: (0, i, 0)),
        pl.BlockSpec((BLK, D), lambda i: (i, 0)),
        pl.BlockSpec((BLK, D), lambda i: (i, 0)),
        pl.BlockSpec((4 * D, D), lambda i: (0, 0)),
        pl.BlockSpec((4 * D, D), lambda i: (0, 0)),
        pl.BlockSpec((1, 4 * D), lambda i: (0, 0)),
    ],
    out_specs=[
        pl.BlockSpec((BLK, D), lambda i: (i, 0)),
        pl.BlockSpec((BLK, D), lambda i: (i, 0)),
    ],
    out_shape=[
        jax.ShapeDtypeStruct((V, D), jnp.float32),
        jax.ShapeDtypeStruct((V, D), jnp.float32),
    ],
)


@jax.jit
def kernel(x_e, edge_index, h_v, c_v, v_batch, W_ih, W_hh, b_ih, b_hh):
    del v_batch
    src = edge_index[0].astype(jnp.int32)
    dst = edge_index[1].astype(jnp.int32)
    zeros = jnp.zeros((VP, D), jnp.float32)
    partials = _scatter_add_sc(x_e, src, dst, zeros)
    bias = (b_ih + b_hh).reshape(1, 4 * D)
    h_new, c_new = _lstm_call(partials, h_v, c_v, W_ih, W_hh, bias)
    return (h_new, c_new)


# speedup vs baseline: 9.2816x; 1.0491x over previous
import functools

import jax
import jax.numpy as jnp
from jax import lax
from jax.experimental import pallas as pl
from jax.experimental.pallas import tpu as pltpu
from jax.experimental.pallas import tpu_sc as plsc

V = 10000
E = 320000
D = 128

NC = 2
NS = 16
NW = NC * NS

CHUNK = 128
NCH = E // CHUNK
NSLOT = 80
NBUF = 2
VP = 10240
RPT = VP // NS

_mesh = plsc.VectorSubcoreMesh(core_axis_name="c", subcore_axis_name="s")


@functools.partial(
    pl.kernel,
    out_type=jax.ShapeDtypeStruct((NC, VP, D), jnp.float32),
    mesh=_mesh,
    scratch_types=(
        [pltpu.VMEM((CHUNK, D), jnp.float32) for _ in range(NBUF)]
        + [pltpu.VMEM((2, CHUNK), jnp.int32) for _ in range(NBUF)]
        + [pltpu.VMEM_SHARED((VP, D), jnp.float32)]
        + [pltpu.SemaphoreType.DMA for _ in range(2 * NBUF)]
    ),
)
def _scatter_add_sc(x_hbm, idx_hbm, zeros_hbm, out_hbm, *scratch):
    xb = scratch[0:NBUF]
    ib = scratch[NBUF:2 * NBUF]
    msg_sh = scratch[2 * NBUF]
    lsem = scratch[2 * NBUF + 1:2 * NBUF + 1 + NBUF]
    ssem = scratch[2 * NBUF + 1 + NBUF:2 * NBUF + 1 + 2 * NBUF]

    c = lax.axis_index("c")
    s = lax.axis_index("s")
    w = c * NS + s

    def issue_loads(b, chid):
        pltpu.async_copy(x_hbm.at[chid], xb[b], lsem[b])
        pltpu.async_copy(idx_hbm.at[chid], ib[b], lsem[b])

    def wait_loads(b):
        pltpu.make_async_copy(x_hbm.at[0], xb[b], lsem[b]).wait()
        pltpu.make_async_copy(idx_hbm.at[0], ib[b], lsem[b]).wait()

    pltpu.sync_copy(zeros_hbm.at[pl.ds(s * RPT, RPT)],
                    msg_sh.at[pl.ds(s * RPT, RPT)])
    plsc.subcore_barrier()

    for b in range(NBUF):
        issue_loads(b, w + NW * b)

    def group(g, carry):
        for b in range(NBUF):
            j = NBUF * g + b
            chid = w + NW * j

            @pl.when(chid < NCH)
            def _():
                wait_loads(b)
                pltpu.async_copy(xb[b], msg_sh.at[ib[b].at[0]], ssem[b], add=True)
                pltpu.async_copy(xb[b], msg_sh.at[ib[b].at[1]], ssem[b], add=True)
                pltpu.make_async_copy(xb[b], msg_sh.at[ib[b].at[0]], ssem[b]).wait()
                pltpu.make_async_copy(xb[b], msg_sh.at[ib[b].at[1]], ssem[b]).wait()

            @pl.when(chid + NW * NBUF < NCH)
            def _():
                issue_loads(b, chid + NW * NBUF)
        return carry

    lax.fori_loop(0, NSLOT // NBUF, group, 0)
    plsc.subcore_barrier()

    pltpu.sync_copy(msg_sh.at[pl.ds(s * RPT, RPT)],
                    out_hbm.at[c, pl.ds(s * RPT, RPT)])


BLK = 400


def _lstm_body(p_ref, h_ref, c_ref, wih_ref, whh_ref, b_ref, ho_ref, co_ref):
    msg = p_ref[0] + p_ref[1]
    h = h_ref[...]
    gates = lax.dot_general(msg, wih_ref[...], (((1,), (1,)), ((), ())),
                            preferred_element_type=jnp.float32)
    gates = gates + lax.dot_general(h, whh_ref[...], (((1,), (1,)), ((), ())),
                                    preferred_element_type=jnp.float32)
    gates = gates + b_ref[...]
    i = jax.nn.sigmoid(gates[:, 0 * D:1 * D])
    f = jax.nn.sigmoid(gates[:, 1 * D:2 * D])
    g = jnp.tanh(gates[:, 2 * D:3 * D])
    o = jax.nn.sigmoid(gates[:, 3 * D:4 * D])
    c_new = f * c_ref[...] + i * g
    ho_ref[...] = o * jnp.tanh(c_new)
    co_ref[...] = c_new


_lstm_call = pl.pallas_call(
    _lstm_body,
    grid=(V // BLK,),
    in_specs=[
        pl.BlockSpec((NC, BLK, D), lambda i: (0, i, 0)),
        pl.BlockSpec((BLK, D), lambda i: (i, 0)),
        pl.BlockSpec((BLK, D), lambda i: (i, 0)),
        pl.BlockSpec((4 * D, D), lambda i: (0, 0)),
        pl.BlockSpec((4 * D, D), lambda i: (0, 0)),
        pl.BlockSpec((1, 4 * D), lambda i: (0, 0)),
    ],
    out_specs=[
        pl.BlockSpec((BLK, D), lambda i: (i, 0)),
        pl.BlockSpec((BLK, D), lambda i: (i, 0)),
    ],
    out_shape=[
        jax.ShapeDtypeStruct((V, D), jnp.float32),
        jax.ShapeDtypeStruct((V, D), jnp.float32),
    ],
)


@jax.jit
def kernel(x_e, edge_index, h_v, c_v, v_batch, W_ih, W_hh, b_ih, b_hh):
    del v_batch
    x3 = x_e.reshape(NCH, CHUNK, D)
    idx3 = edge_index.astype(jnp.int32).reshape(2, NCH, CHUNK).transpose(1, 0, 2)
    zeros = jnp.zeros((VP, D), jnp.float32)
    partials = _scatter_add_sc(x3, idx3, zeros)
    bias = (b_ih + b_hh).reshape(1, 4 * D)
    h_new, c_new = _lstm_call(partials, h_v, c_v, W_ih, W_hh, bias)
    return (h_new, c_new)


# speedup vs baseline: 9.2879x; 1.0007x over previous
import functools

import jax
import jax.numpy as jnp
from jax import lax
from jax.experimental import pallas as pl
from jax.experimental.pallas import tpu as pltpu
from jax.experimental.pallas import tpu_sc as plsc

V = 10000
E = 320000
D = 128

NC = 2
NS = 16
NW = NC * NS

CHUNK = 128
NCH = E // CHUNK
NSLOT = 80
NBUF = 2
VP = 10240
RPT = VP // NS

_mesh = plsc.VectorSubcoreMesh(core_axis_name="c", subcore_axis_name="s")


@functools.partial(
    pl.kernel,
    out_type=jax.ShapeDtypeStruct((NC, VP, D), jnp.float32),
    mesh=_mesh,
    scratch_types=(
        [pltpu.VMEM((CHUNK, D), jnp.float32) for _ in range(NBUF)]
        + [pltpu.VMEM((2, CHUNK), jnp.int32) for _ in range(NBUF)]
        + [pltpu.VMEM_SHARED((VP, D), jnp.float32)]
        + [pltpu.SemaphoreType.DMA for _ in range(2 * NBUF)]
    ),
)
def _scatter_add_sc(x_hbm, idx_hbm, zeros_hbm, out_hbm, *scratch):
    xb = scratch[0:NBUF]
    ib = scratch[NBUF:2 * NBUF]
    msg_sh = scratch[2 * NBUF]
    lsem = scratch[2 * NBUF + 1:2 * NBUF + 1 + NBUF]
    ssem = scratch[2 * NBUF + 1 + NBUF:2 * NBUF + 1 + 2 * NBUF]

    c = lax.axis_index("c")
    s = lax.axis_index("s")
    w = c * NS + s

    def issue_loads(b, chid):
        pltpu.async_copy(x_hbm.at[chid], xb[b], lsem[b])
        pltpu.async_copy(idx_hbm.at[chid], ib[b], lsem[b])

    def wait_loads(b):
        pltpu.make_async_copy(x_hbm.at[0], xb[b], lsem[b]).wait()
        pltpu.make_async_copy(idx_hbm.at[0], ib[b], lsem[b]).wait()

    pltpu.sync_copy(zeros_hbm.at[pl.ds(s * RPT, RPT)],
                    msg_sh.at[pl.ds(s * RPT, RPT)])
    plsc.subcore_barrier()

    def drain_scatters(b):
        pltpu.make_async_copy(xb[b], msg_sh.at[ib[b].at[0]], ssem[b]).wait()
        pltpu.make_async_copy(xb[b], msg_sh.at[ib[b].at[1]], ssem[b]).wait()

    issue_loads(0, w)

    def group(g, carry):
        for p in range(NBUF):
            j = NBUF * g + p
            chid = w + NW * j
            q = (p + 1) % NBUF

            @pl.when(chid < NCH)
            def _():
                wait_loads(p)
                pltpu.async_copy(xb[p], msg_sh.at[ib[p].at[0]], ssem[p], add=True)
                pltpu.async_copy(xb[p], msg_sh.at[ib[p].at[1]], ssem[p], add=True)

            @pl.when(jnp.logical_and(chid >= NW, chid - NW < NCH))
            def _():
                drain_scatters(q)

            @pl.when(chid + NW < NCH)
            def _():
                issue_loads(q, chid + NW)
        return carry

    lax.fori_loop(0, NSLOT // NBUF, group, 0)
    plsc.subcore_barrier()

    pltpu.sync_copy(msg_sh.at[pl.ds(s * RPT, RPT)],
                    out_hbm.at[c, pl.ds(s * RPT, RPT)])


BLK = 400


def _lstm_body(p_ref, h_ref, c_ref, wih_ref, whh_ref, b_ref, ho_ref, co_ref):
    msg = p_ref[0] + p_ref[1]
    h = h_ref[...]
    gates = lax.dot_general(msg, wih_ref[...], (((1,), (1,)), ((), ())),
                            preferred_element_type=jnp.float32)
    gates = gates + lax.dot_general(h, whh_ref[...], (((1,), (1,)), ((), ())),
                                    preferred_element_type=jnp.float32)
    gates = gates + b_ref[...]
    i = jax.nn.sigmoid(gates[:, 0 * D:1 * D])
    f = jax.nn.sigmoid(gates[:, 1 * D:2 * D])
    g = jnp.tanh(gates[:, 2 * D:3 * D])
    o = jax.nn.sigmoid(gates[:, 3 * D:4 * D])
    c_new = f * c_ref[...] + i * g
    ho_ref[...] = o * jnp.tanh(c_new)
    co_ref[...] = c_new


_lstm_call = pl.pallas_call(
    _lstm_body,
    grid=(V // BLK,),
    in_specs=[
        pl.BlockSpec((NC, BLK, D), lambda i: (0, i, 0)),
        pl.BlockSpec((BLK, D), lambda i: (i, 0)),
        pl.BlockSpec((BLK, D), lambda i: (i, 0)),
        pl.BlockSpec((4 * D, D), lambda i: (0, 0)),
        pl.BlockSpec((4 * D, D), lambda i: (0, 0)),
        pl.BlockSpec((1, 4 * D), lambda i: (0, 0)),
    ],
    out_specs=[
        pl.BlockSpec((BLK, D), lambda i: (i, 0)),
        pl.BlockSpec((BLK, D), lambda i: (i, 0)),
    ],
    out_shape=[
        jax.ShapeDtypeStruct((V, D), jnp.float32),
        jax.ShapeDtypeStruct((V, D), jnp.float32),
    ],
)


@jax.jit
def kernel(x_e, edge_index, h_v, c_v, v_batch, W_ih, W_hh, b_ih, b_hh):
    del v_batch
    x3 = x_e.reshape(NCH, CHUNK, D)
    idx3 = edge_index.astype(jnp.int32).reshape(2, NCH, CHUNK).transpose(1, 0, 2)
    zeros = jnp.zeros((VP, D), jnp.float32)
    partials = _scatter_add_sc(x3, idx3, zeros)
    bias = (b_ih + b_hh).reshape(1, 4 * D)
    h_new, c_new = _lstm_call(partials, h_v, c_v, W_ih, W_hh, bias)
    return (h_new, c_new)


# speedup vs baseline: 9.7052x; 1.0449x over previous
import functools

import jax
import jax.numpy as jnp
from jax import lax
from jax.experimental import pallas as pl
from jax.experimental.pallas import tpu as pltpu
from jax.experimental.pallas import tpu_sc as plsc

V = 10000
E = 320000
D = 128

NC = 2
NS = 16
NW = NC * NS

CHUNK = 128
NCH = E // CHUNK
NSLOT = 80
NBUF = 2
VP = 10240
RPT = VP // NS

_mesh = plsc.VectorSubcoreMesh(core_axis_name="c", subcore_axis_name="s")


@functools.partial(
    pl.kernel,
    out_type=jax.ShapeDtypeStruct((NC, VP, D), jnp.float32),
    mesh=_mesh,
    scratch_types=(
        [pltpu.VMEM((CHUNK, D), jnp.float32) for _ in range(NBUF)]
        + [pltpu.VMEM((2, CHUNK), jnp.int32) for _ in range(NBUF)]
        + [pltpu.VMEM_SHARED((VP, D), jnp.float32)]
        + [pltpu.SemaphoreType.DMA for _ in range(2 * NBUF)]
    ),
)
def _scatter_add_sc(x_hbm, idx_hbm, zeros_hbm, out_hbm, *scratch):
    xb = scratch[0:NBUF]
    ib = scratch[NBUF:2 * NBUF]
    msg_sh = scratch[2 * NBUF]
    lsem = scratch[2 * NBUF + 1:2 * NBUF + 1 + NBUF]
    ssem = scratch[2 * NBUF + 1 + NBUF:2 * NBUF + 1 + 2 * NBUF]

    c = lax.axis_index("c")
    s = lax.axis_index("s")
    w = c * NS + s

    def issue_loads(b, chid):
        pltpu.async_copy(x_hbm.at[chid], xb[b], lsem[b])
        pltpu.async_copy(idx_hbm.at[chid], ib[b], lsem[b])

    def wait_loads(b):
        pltpu.make_async_copy(x_hbm.at[0], xb[b], lsem[b]).wait()
        pltpu.make_async_copy(idx_hbm.at[0], ib[b], lsem[b]).wait()

    pltpu.sync_copy(zeros_hbm.at[pl.ds(s * RPT, RPT)],
                    msg_sh.at[pl.ds(s * RPT, RPT)])
    plsc.subcore_barrier()

    def drain_scatters(b):
        pltpu.make_async_copy(xb[b], msg_sh.at[ib[b].at[0]], ssem[b]).wait()
        pltpu.make_async_copy(xb[b], msg_sh.at[ib[b].at[1]], ssem[b]).wait()

    issue_loads(0, w)

    def group(g, carry):
        for p in range(NBUF):
            j = NBUF * g + p
            chid = w + NW * j
            q = (p + 1) % NBUF

            @pl.when(chid < NCH)
            def _():
                wait_loads(p)
                pltpu.async_copy(xb[p], msg_sh.at[ib[p].at[0]], ssem[p], add=True)
                pltpu.async_copy(xb[p], msg_sh.at[ib[p].at[1]], ssem[p], add=True)

            @pl.when(jnp.logical_and(chid >= NW, chid - NW < NCH))
            def _():
                drain_scatters(q)

            @pl.when(chid + NW < NCH)
            def _():
                issue_loads(q, chid + NW)
        return carry

    lax.fori_loop(0, NSLOT // NBUF, group, 0)
    plsc.subcore_barrier()

    pltpu.sync_copy(msg_sh.at[pl.ds(s * RPT, RPT)],
                    out_hbm.at[c, pl.ds(s * RPT, RPT)])


BLK = 1000


def _lstm_body(p_ref, h_ref, c_ref, wih_ref, whh_ref, b_ref, ho_ref, co_ref):
    msg = p_ref[0] + p_ref[1]
    h = h_ref[...]
    gates = lax.dot_general(msg, wih_ref[...], (((1,), (1,)), ((), ())),
                            preferred_element_type=jnp.float32)
    gates = gates + lax.dot_general(h, whh_ref[...], (((1,), (1,)), ((), ())),
                                    preferred_element_type=jnp.float32)
    gates = gates + b_ref[...]
    i = jax.nn.sigmoid(gates[:, 0 * D:1 * D])
    f = jax.nn.sigmoid(gates[:, 1 * D:2 * D])
    g = jnp.tanh(gates[:, 2 * D:3 * D])
    o = jax.nn.sigmoid(gates[:, 3 * D:4 * D])
    c_new = f * c_ref[...] + i * g
    ho_ref[...] = o * jnp.tanh(c_new)
    co_ref[...] = c_new


_lstm_call = pl.pallas_call(
    _lstm_body,
    grid=(V // BLK,),
    in_specs=[
        pl.BlockSpec((NC, BLK, D), lambda i: (0, i, 0)),
        pl.BlockSpec((BLK, D), lambda i: (i, 0)),
        pl.BlockSpec((BLK, D), lambda i: (i, 0)),
        pl.BlockSpec((4 * D, D), lambda i: (0, 0)),
        pl.BlockSpec((4 * D, D), lambda i: (0, 0)),
        pl.BlockSpec((1, 4 * D), lambda i: (0, 0)),
    ],
    out_specs=[
        pl.BlockSpec((BLK, D), lambda i: (i, 0)),
        pl.BlockSpec((BLK, D), lambda i: (i, 0)),
    ],
    out_shape=[
        jax.ShapeDtypeStruct((V, D), jnp.float32),
        jax.ShapeDtypeStruct((V, D), jnp.float32),
    ],
)


@jax.jit
def kernel(x_e, edge_index, h_v, c_v, v_batch, W_ih, W_hh, b_ih, b_hh):
    del v_batch
    x3 = x_e.reshape(NCH, CHUNK, D)
    idx3 = edge_index.astype(jnp.int32).reshape(2, NCH, CHUNK).transpose(1, 0, 2)
    zeros = jnp.zeros((VP, D), jnp.float32)
    partials = _scatter_add_sc(x3, idx3, zeros)
    bias = (b_ih + b_hh).reshape(1, 4 * D)
    h_new, c_new = _lstm_call(partials, h_v, c_v, W_ih, W_hh, bias)
    return (h_new, c_new)


# speedup vs baseline: 9.8265x; 1.0125x over previous
import functools

import jax
import jax.numpy as jnp
from jax import lax
from jax.experimental import pallas as pl
from jax.experimental.pallas import tpu as pltpu
from jax.experimental.pallas import tpu_sc as plsc

V = 10000
E = 320000
D = 128

NC = 2
NS = 16
NW = NC * NS

CHUNK = 128
NCH = E // CHUNK
NSLOT = 80
NBUF = 2
VP = 10240
RPT = VP // NS

_mesh = plsc.VectorSubcoreMesh(core_axis_name="c", subcore_axis_name="s")


@functools.partial(
    pl.kernel,
    out_type=jax.ShapeDtypeStruct((NC, VP, D), jnp.float32),
    mesh=_mesh,
    scratch_types=(
        [pltpu.VMEM((CHUNK, D), jnp.float32) for _ in range(NBUF)]
        + [pltpu.VMEM((2, CHUNK), jnp.int32) for _ in range(NBUF)]
        + [pltpu.VMEM_SHARED((VP, D), jnp.float32)]
        + [pltpu.SemaphoreType.DMA for _ in range(2 * NBUF)]
    ),
)
def _scatter_add_sc(x_hbm, idx_hbm, zeros_hbm, out_hbm, *scratch):
    xb = scratch[0:NBUF]
    ib = scratch[NBUF:2 * NBUF]
    msg_sh = scratch[2 * NBUF]
    lsem = scratch[2 * NBUF + 1:2 * NBUF + 1 + NBUF]
    ssem = scratch[2 * NBUF + 1 + NBUF:2 * NBUF + 1 + 2 * NBUF]

    c = lax.axis_index("c")
    s = lax.axis_index("s")
    w = c * NS + s

    def issue_loads(b, chid):
        pltpu.async_copy(x_hbm.at[chid], xb[b], lsem[b])
        pltpu.async_copy(idx_hbm.at[chid], ib[b], lsem[b])

    def wait_loads(b):
        pltpu.make_async_copy(x_hbm.at[0], xb[b], lsem[b]).wait()
        pltpu.make_async_copy(idx_hbm.at[0], ib[b], lsem[b]).wait()

    def drain_scatters(b):
        pltpu.make_async_copy(xb[b], msg_sh.at[ib[b].at[0]], ssem[b]).wait()
        pltpu.make_async_copy(xb[b], msg_sh.at[ib[b].at[1]], ssem[b]).wait()

    issue_loads(0, w)

    pltpu.sync_copy(zeros_hbm.at[pl.ds(s * RPT, RPT)],
                    msg_sh.at[pl.ds(s * RPT, RPT)])
    plsc.subcore_barrier()

    def group(g, carry):
        for p in range(NBUF):
            j = NBUF * g + p
            chid = w + NW * j
            q = (p + 1) % NBUF

            @pl.when(chid < NCH)
            def _():
                wait_loads(p)
                pltpu.async_copy(xb[p], msg_sh.at[ib[p].at[0]], ssem[p], add=True)
                pltpu.async_copy(xb[p], msg_sh.at[ib[p].at[1]], ssem[p], add=True)

            @pl.when(jnp.logical_and(chid >= NW, chid - NW < NCH))
            def _():
                drain_scatters(q)

            @pl.when(chid + NW < NCH)
            def _():
                issue_loads(q, chid + NW)
        return carry

    lax.fori_loop(0, NSLOT // NBUF, group, 0)
    plsc.subcore_barrier()

    pltpu.sync_copy(msg_sh.at[pl.ds(s * RPT, RPT)],
                    out_hbm.at[c, pl.ds(s * RPT, RPT)])


BLK = 1000


def _lstm_body(p_ref, h_ref, c_ref, wih_ref, whh_ref, b_ref, ho_ref, co_ref):
    msg = p_ref[0] + p_ref[1]
    h = h_ref[...]
    gates = lax.dot_general(msg, wih_ref[...], (((1,), (1,)), ((), ())),
                            preferred_element_type=jnp.float32)
    gates = gates + lax.dot_general(h, whh_ref[...], (((1,), (1,)), ((), ())),
                                    preferred_element_type=jnp.float32)
    gates = gates + b_ref[...]
    i = jax.nn.sigmoid(gates[:, 0 * D:1 * D])
    f = jax.nn.sigmoid(gates[:, 1 * D:2 * D])
    g = jnp.tanh(gates[:, 2 * D:3 * D])
    o = jax.nn.sigmoid(gates[:, 3 * D:4 * D])
    c_new = f * c_ref[...] + i * g
    ho_ref[...] = o * jnp.tanh(c_new)
    co_ref[...] = c_new


_lstm_call = pl.pallas_call(
    _lstm_body,
    grid=(V // BLK,),
    in_specs=[
        pl.BlockSpec((NC, BLK, D), lambda i: (0, i, 0)),
        pl.BlockSpec((BLK, D), lambda i: (i, 0)),
        pl.BlockSpec((BLK, D), lambda i: (i, 0)),
        pl.BlockSpec((4 * D, D), lambda i: (0, 0)),
        pl.BlockSpec((4 * D, D), lambda i: (0, 0)),
        pl.BlockSpec((1, 4 * D), lambda i: (0, 0)),
    ],
    out_specs=[
        pl.BlockSpec((BLK, D), lambda i: (i, 0)),
        pl.BlockSpec((BLK, D), lambda i: (i, 0)),
    ],
    out_shape=[
        jax.ShapeDtypeStruct((V, D), jnp.float32),
        jax.ShapeDtypeStruct((V, D), jnp.float32),
    ],
)


@jax.jit
def kernel(x_e, edge_index, h_v, c_v, v_batch, W_ih, W_hh, b_ih, b_hh):
    del v_batch
    x3 = x_e.reshape(NCH, CHUNK, D)
    idx3 = edge_index.astype(jnp.int32).reshape(2, NCH, CHUNK).transpose(1, 0, 2)
    zeros = jnp.zeros((VP, D), jnp.float32)
    partials = _scatter_add_sc(x3, idx3, zeros)
    bias = (b_ih + b_hh).reshape(1, 4 * D)
    h_new, c_new = _lstm_call(partials, h_v, c_v, W_ih, W_hh, bias)
    return (h_new, c_new)
